# Initial kernel scaffold; baseline (speedup 1.0000x reference)
import jax
import jax.numpy as jnp
from jax.experimental import pallas as pl


def kernel(x, init_x, edge_index, edge_weight, W, b):
    raise NotImplementedError("write your pallas kernel here")


# speedup vs baseline: 4.3440x; 4.3440x over previous
import functools

import jax
import jax.numpy as jnp
from jax import lax
from jax.experimental import pallas as pl
from jax.experimental.pallas import tpu as pltpu
from jax.experimental.pallas import tpu_sc as plsc

_ALPHA = 0.1
_BETA = 0.5

_NC = 2
_NS = 16
_NW = _NC * _NS
_C = 128


def _spmm_body(n_pad, n_groups, lanes,
               x_hbm, src_hbm, dst_hbm, w_hbm, zero_hbm, out_hbm,
               src_v, dst_v, w_v, rows_v, acc_sh, sem):
  cid = lax.axis_index("c")
  sid = lax.axis_index("s")
  wid = cid * _NS + sid
  stripe = n_pad // _NS

  pltpu.sync_copy(zero_hbm.at[pl.ds(sid * stripe, stripe)],
                  acc_sh.at[pl.ds(sid * stripe, stripe)])
  plsc.subcore_barrier()

  pltpu.sync_copy(src_hbm.at[wid], src_v)
  pltpu.sync_copy(dst_hbm.at[wid], dst_v)
  pltpu.sync_copy(w_hbm.at[wid], w_v)

  d = rows_v.shape[1]

  def group(g, carry):
    pltpu.async_copy(x_hbm.at[src_v.at[g]], rows_v, sem).wait()

    def subblock(sb, carry):
      wv = w_v[g, pl.ds(sb * lanes, lanes)]
      for i in range(lanes):
        e_row = sb * lanes + i
        w = wv[i]
        for j in range(d // lanes):
          sl = pl.ds(j * lanes, lanes)
          rows_v[e_row, sl] = rows_v[e_row, sl] * w
      return carry

    lax.fori_loop(0, _C // lanes, subblock, carry)

    pltpu.sync_copy(rows_v, acc_sh.at[dst_v.at[g]], add=True)
    return carry

  lax.fori_loop(0, n_groups, group, 0)
  plsc.subcore_barrier()

  pltpu.sync_copy(acc_sh.at[pl.ds(sid * stripe, stripe)],
                  out_hbm.at[cid, pl.ds(sid * stripe, stripe)])


def _dense_body(p0_ref, p1_ref, ix_ref, wt_ref, b_ref, o_ref):
  hid = (1.0 - _ALPHA) * (p0_ref[...] + p1_ref[...]) + _ALPHA * ix_ref[...]
  lin = jnp.dot(hid, wt_ref[...], preferred_element_type=jnp.float32)
  o_ref[...] = _BETA * (lin + b_ref[...]) + (1.0 - _BETA) * hid


def kernel(x, init_x, edge_index, edge_weight, W, b):
  n, d = x.shape
  e = edge_weight.shape[0]
  n_groups = -(-e // (_NW * _C))
  e_pad = _NW * n_groups * _C

  src = edge_index[0]
  dst = edge_index[1]
  ew = edge_weight
  if e_pad != e:
    pad = e_pad - e
    src = jnp.concatenate([src, jnp.zeros((pad,), src.dtype)])
    dst = jnp.concatenate([dst, jnp.zeros((pad,), dst.dtype)])
    ew = jnp.concatenate([ew, jnp.zeros((pad,), ew.dtype)])
  src = src.reshape(_NW, n_groups, _C)
  dst = dst.reshape(_NW, n_groups, _C)
  ew = ew.reshape(_NW, n_groups, _C)
  n_pad = -(-n // 128) * 128
  zero_nd = jnp.zeros((n_pad, d), x.dtype)

  info = plsc.get_sparse_core_info()
  lanes = info.num_lanes
  mesh = plsc.VectorSubcoreMesh(core_axis_name="c", subcore_axis_name="s")
  spmm = pl.kernel(
      functools.partial(_spmm_body, n_pad, n_groups, lanes),
      out_type=jax.ShapeDtypeStruct((_NC, n_pad, d), jnp.float32),
      mesh=mesh,
      scratch_types=[
          pltpu.VMEM((n_groups, _C), jnp.int32),
          pltpu.VMEM((n_groups, _C), jnp.int32),
          pltpu.VMEM((n_groups, _C), jnp.float32),
          pltpu.VMEM((_C, d), jnp.float32),
          pltpu.VMEM_SHARED((n_pad, d), jnp.float32),
          pltpu.SemaphoreType.DMA,
      ],
  )
  partial = spmm(x, src, dst, ew, zero_nd)

  bn = 1000
  wt = W.T
  b2 = b.reshape(1, d)
  return pl.pallas_call(
      _dense_body,
      grid=(n // bn,),
      in_specs=[
          pl.BlockSpec((bn, d), lambda i: (i, 0)),
          pl.BlockSpec((bn, d), lambda i: (i, 0)),
          pl.BlockSpec((bn, d), lambda i: (i, 0)),
          pl.BlockSpec((d, d), lambda i: (0, 0)),
          pl.BlockSpec((1, d), lambda i: (0, 0)),
      ],
      out_specs=pl.BlockSpec((bn, d), lambda i: (i, 0)),
      out_shape=jax.ShapeDtypeStruct((n, d), jnp.float32),
  )(partial[0, :n], partial[1, :n], init_x, wt, b2)
